# baseline (speedup 1.0000x reference)
import functools
import math

import jax
import jax.numpy as jnp
import numpy as np
from jax import lax
from jax.experimental import pallas as pl
from jax.experimental.pallas import tpu as pltpu
from jax.experimental.pallas import tpu_sc as plsc

_N = 10000
_E = 320000
_F = 128
_EV = 16
_XD = _F + _EV
_NC, _NS = 2, 16
_NW = _NC * _NS
_CHUNK = 80
_PER_W = _E // _NW
_REPEATS = (1, 3, 5, 7)
_OFFS = (0, 1, 4, 9)


def _sc_gather_body(x_hbm, s_hbm, r_hbm, outs_hbm, outr_hbm, idx_v, rows_v):
    wid = lax.axis_index("s") * _NC + lax.axis_index("c")
    base = wid * _PER_W

    @pl.loop(0, _PER_W // _CHUNK)
    def _(i):
        off = base + i * _CHUNK
        pltpu.sync_copy(s_hbm.at[pl.ds(off, _CHUNK)], idx_v)
        pltpu.sync_copy(x_hbm.at[idx_v], rows_v)
        pltpu.sync_copy(rows_v, outs_hbm.at[pl.ds(off, _CHUNK)])
        pltpu.sync_copy(r_hbm.at[pl.ds(off, _CHUNK)], idx_v)
        pltpu.sync_copy(x_hbm.at[idx_v], rows_v)
        pltpu.sync_copy(rows_v, outr_hbm.at[pl.ds(off, _CHUNK)])


def _sc_gather(x, senders, receivers):
    mesh = plsc.VectorSubcoreMesh(core_axis_name="c", subcore_axis_name="s")
    f = pl.kernel(
        _sc_gather_body,
        out_type=[
            jax.ShapeDtypeStruct((_E, _XD), jnp.float32),
            jax.ShapeDtypeStruct((_E, _XD), jnp.float32),
        ],
        mesh=mesh,
        scratch_types=[
            pltpu.VMEM((_CHUNK,), jnp.int32),
            pltpu.VMEM((_CHUNK, _XD), jnp.float32),
        ],
        compiler_params=pltpu.CompilerParams(use_tc_tiling_on_sc=False),
    )
    return f(x, senders, receivers)


_EPC = _E // _NC
_PW2 = _EPC // _NS
_ZROWS = 624
_ZTAIL = _N - _ZROWS * _NS


def _sc_scatter_body(scaled_hbm, recv_hbm, zeros_hbm, out_hbm, idx_v, rows_v, acc_sh):
    c = lax.axis_index("c")
    s = lax.axis_index("s")
    z0 = s * _ZROWS
    pltpu.sync_copy(zeros_hbm.at[pl.ds(0, _ZROWS)], acc_sh.at[pl.ds(z0, _ZROWS)])

    @pl.when(s == _NS - 1)
    def _():
        pltpu.sync_copy(
            zeros_hbm.at[pl.ds(0, _ZTAIL)],
            acc_sh.at[pl.ds(_ZROWS * _NS, _ZTAIL)],
        )

    plsc.subcore_barrier()

    base = c * _EPC + s * _PW2

    @pl.loop(0, _PW2 // _CHUNK)
    def _(i):
        off = base + i * _CHUNK
        pltpu.sync_copy(recv_hbm.at[pl.ds(off, _CHUNK)], idx_v)
        pltpu.sync_copy(scaled_hbm.at[pl.ds(off, _CHUNK)], rows_v)
        pltpu.sync_copy(rows_v, acc_sh.at[idx_v], add=True)

    plsc.subcore_barrier()

    pltpu.sync_copy(acc_sh.at[pl.ds(z0, _ZROWS)], out_hbm.at[c, pl.ds(z0, _ZROWS)])

    @pl.when(s == _NS - 1)
    def _():
        pltpu.sync_copy(
            acc_sh.at[pl.ds(_ZROWS * _NS, _ZTAIL)],
            out_hbm.at[c, pl.ds(_ZROWS * _NS, _ZTAIL)],
        )


def _sc_scatter(scaled, receivers, zeros):
    mesh = plsc.VectorSubcoreMesh(core_axis_name="c", subcore_axis_name="s")
    f = pl.kernel(
        _sc_scatter_body,
        out_type=jax.ShapeDtypeStruct((_NC, _N, _XD), jnp.float32),
        mesh=mesh,
        scratch_types=[
            pltpu.VMEM((_CHUNK,), jnp.int32),
            pltpu.VMEM((_CHUNK, _XD), jnp.float32),
            pltpu.VMEM_SHARED((_N, _XD), jnp.float32),
        ],
        compiler_params=pltpu.CompilerParams(use_tc_tiling_on_sc=False),
    )
    return f(scaled, receivers, zeros)


_BE = 512


def _tc_edge_body(xs_ref, xr_ref, rbf_ref, sh_ref, cut_ref,
                  qbd_ref, kbd_ref, vbd_ref, w1_ref, b1_ref, w2_ref, b2_ref,
                  hc_ref, ec_ref, sc_ref, out_ref):
    f32 = jnp.float32
    bf16 = jnp.bfloat16
    xs = xs_ref[...]
    xr = xr_ref[...]
    xs_inv = xs[:, :_F].astype(bf16)
    xr_inv = xr[:, :_F].astype(bf16)

    dev = xs[:, _F:] - xr[:, _F:]
    inv_c = jnp.dot((dev * dev).astype(bf16), sc_ref[...], preferred_element_type=f32)

    fin = jnp.concatenate([rbf_ref[...].astype(bf16), inv_c.astype(bf16)], axis=1)
    h = jnp.dot(fin, w1_ref[...], preferred_element_type=f32) + b1_ref[...]
    h = jax.nn.silu(h)
    fwcat = jnp.dot(h.astype(bf16), w2_ref[...], preferred_element_type=f32) + b2_ref[...]

    qcat = jax.nn.silu(jnp.dot(xr_inv, qbd_ref[...], preferred_element_type=f32))
    kcat = jax.nn.silu(jnp.dot(xs_inv, kbd_ref[...], preferred_element_type=f32))
    v = jnp.dot(xs_inv, vbd_ref[...], preferred_element_type=f32)

    prod = qcat * kcat * fwcat
    alpha = jnp.dot(prod.astype(bf16), hc_ref[...], preferred_element_type=f32)
    coef = cut_ref[...] * alpha
    coef_exp = jnp.dot(coef.astype(bf16), ec_ref[...], preferred_element_type=f32)

    out_ref[:, :_F] = coef_exp[:, :_F] * v
    out_ref[:, _F:] = coef_exp[:, _F:] * sh_ref[...]


def _tc_edge(xg_s, xg_r, rbf, sh, cut, qbd, kbd, vbd, w1, b1, w2, b2, hc, ec, sc):
    grid = (_E // _BE,)
    blk = lambda *shape: pl.BlockSpec(shape, lambda i: (i,) + (0,) * (len(shape) - 1))
    full = lambda *shape: pl.BlockSpec(shape, lambda i: (0,) * len(shape))
    return pl.pallas_call(
        _tc_edge_body,
        grid=grid,
        in_specs=[
            blk(_BE, _XD), blk(_BE, _XD), blk(_BE, 32), blk(_BE, _EV), blk(_BE, 1),
            full(_F, 256), full(_F, 256), full(_F, _F), full(36, _F), full(1, _F),
            full(_F, 256), full(1, 256), full(256, 12), full(12, _XD), full(_EV, 4),
        ],
        out_specs=blk(_BE, _XD),
        out_shape=jax.ShapeDtypeStruct((_E, _XD), jnp.float32),
    )(xg_s, xg_r, rbf, sh, cut, qbd, kbd, vbd, w1, b1, w2, b2, hc, ec, sc)


_BN = 2000


def _tc_reduce_body(p_ref, inv_ref, ev_ref):
    s = p_ref[0] + p_ref[1]
    inv_ref[...] = s[:, :_F]
    ev_ref[...] = s[:, _F:]


def _tc_reduce(partials):
    return pl.pallas_call(
        _tc_reduce_body,
        grid=(_N // _BN,),
        in_specs=[pl.BlockSpec((2, _BN, _XD), lambda i: (0, i, 0))],
        out_specs=[
            pl.BlockSpec((_BN, _F), lambda i: (i, 0)),
            pl.BlockSpec((_BN, _EV), lambda i: (i, 0)),
        ],
        out_shape=[
            jax.ShapeDtypeStruct((_N, _F), jnp.float32),
            jax.ShapeDtypeStruct((_N, _EV), jnp.float32),
        ],
    )(partials)


def _block_diag(w):
    h, d, e = w.shape
    eye = jnp.eye(h, dtype=w.dtype)
    return jnp.einsum("hde,hg->hdge", w, eye).reshape(h * d, h * e)


def _static_mats():
    hc = np.zeros((256, 12), np.float32)
    for f in range(128):
        hc[f, f // 16] = 1.0
    for f in range(128):
        hc[128 + f, 8 + f // 32] = 1.0
    ec = np.zeros((12, _XD), np.float32)
    for h in range(8):
        ec[h, 16 * h:16 * h + 16] = 1.0 / 4.0
    for l in range(4):
        ec[8 + l, _F + _OFFS[l]:_F + _OFFS[l] + _REPEATS[l]] = 1.0 / math.sqrt(32.0)
    sc = np.zeros((_EV, 4), np.float32)
    for l in range(4):
        sc[_OFFS[l]:_OFFS[l] + _REPEATS[l], l] = 1.0
    return jnp.asarray(hc), jnp.asarray(ec), jnp.asarray(sc)


def kernel(inv_features, ev_features, rbf, senders, receivers, sh_vectors, cutoffs,
           W_q_inv, W_k_inv, W_v_inv, W_q_ev, W_k_ev,
           W1_inv, b1_inv, W2_inv, b2_inv, W1_ev, b1_ev, W2_ev, b2_ev):
    x = jnp.concatenate([inv_features, ev_features], axis=1)
    senders = senders.astype(jnp.int32)
    receivers = receivers.astype(jnp.int32)

    qbd = jnp.concatenate([_block_diag(W_q_inv), _block_diag(W_q_ev)], axis=1)
    kbd = jnp.concatenate([_block_diag(W_k_inv), _block_diag(W_k_ev)], axis=1)
    vbd = _block_diag(W_v_inv)
    w1 = jnp.concatenate([W1_inv, W1_ev], axis=1)
    b1 = jnp.concatenate([b1_inv, b1_ev]).reshape(1, -1)
    zero64 = jnp.zeros((64, _F), jnp.float32)
    w2 = jnp.concatenate(
        [jnp.concatenate([W2_inv, zero64], axis=1),
         jnp.concatenate([zero64, W2_ev], axis=1)], axis=0)
    b2 = jnp.concatenate([b2_inv, b2_ev]).reshape(1, -1)
    hc, ec, sc = _static_mats()
    bf16 = jnp.bfloat16
    qbd, kbd, vbd, w1, w2, hc, ec, sc = (
        a.astype(bf16) for a in (qbd, kbd, vbd, w1, w2, hc, ec, sc))

    xg_s, xg_r = _sc_gather(x, senders, receivers)
    scaled = _tc_edge(xg_s, xg_r, rbf, sh_vectors, cutoffs,
                      qbd, kbd, vbd, w1, b1, w2, b2, hc, ec, sc)
    zeros = jnp.zeros((_ZROWS, _XD), jnp.float32)
    partials = _sc_scatter(scaled, receivers, zeros)
    d_att_inv, d_att_ev = _tc_reduce(partials)
    return (d_att_inv, d_att_ev)


# speedup vs baseline: 1.2340x; 1.2340x over previous
import functools
import math

import jax
import jax.numpy as jnp
import numpy as np
from jax import lax
from jax.experimental import pallas as pl
from jax.experimental.pallas import tpu as pltpu
from jax.experimental.pallas import tpu_sc as plsc

_N = 10000
_E = 320000
_F = 128
_EV = 16
_XD = _F + _EV
_NC, _NS = 2, 16
_NW = _NC * _NS
_CHUNK = 80
_NCK = 5
_CE = _E // _NCK
_GPW = _CE // _NW
_REPEATS = (1, 3, 5, 7)
_OFFS = (0, 1, 4, 9)


def _sc_gather_body(x_hbm, s_hbm, r_hbm, outs_hbm, outr_hbm, idx_v, rows_v):
    wid = lax.axis_index("s") * _NC + lax.axis_index("c")
    base = wid * _GPW

    @pl.loop(0, _GPW // _CHUNK)
    def _(i):
        off = base + i * _CHUNK
        pltpu.sync_copy(s_hbm.at[pl.ds(off, _CHUNK)], idx_v)
        pltpu.sync_copy(x_hbm.at[idx_v], rows_v)
        pltpu.sync_copy(rows_v, outs_hbm.at[pl.ds(off, _CHUNK)])
        pltpu.sync_copy(r_hbm.at[pl.ds(off, _CHUNK)], idx_v)
        pltpu.sync_copy(x_hbm.at[idx_v], rows_v)
        pltpu.sync_copy(rows_v, outr_hbm.at[pl.ds(off, _CHUNK)])


def _sc_gather(x, senders, receivers):
    mesh = plsc.VectorSubcoreMesh(core_axis_name="c", subcore_axis_name="s")
    f = pl.kernel(
        _sc_gather_body,
        out_type=[
            jax.ShapeDtypeStruct((_CE, _XD), jnp.float32),
            jax.ShapeDtypeStruct((_CE, _XD), jnp.float32),
        ],
        mesh=mesh,
        scratch_types=[
            pltpu.VMEM((_CHUNK,), jnp.int32),
            pltpu.VMEM((_CHUNK, _XD), jnp.float32),
        ],
        compiler_params=pltpu.CompilerParams(use_tc_tiling_on_sc=False),
    )
    return f(x, senders, receivers)


_CPC = _CE // _NC
_PW2 = _CPC // _NS
_ZROWS = 624
_ZTAIL = _N - _ZROWS * _NS


def _sc_scatter_body(s0, s1, s2, s3, s4, recv_hbm, zeros_hbm, out_hbm,
                     idx_v, rows_v, acc_sh):
    c = lax.axis_index("c")
    s = lax.axis_index("s")
    z0 = s * _ZROWS
    pltpu.sync_copy(zeros_hbm.at[pl.ds(0, _ZROWS)], acc_sh.at[pl.ds(z0, _ZROWS)])

    @pl.when(s == _NS - 1)
    def _():
        pltpu.sync_copy(
            zeros_hbm.at[pl.ds(0, _ZTAIL)],
            acc_sh.at[pl.ds(_ZROWS * _NS, _ZTAIL)],
        )

    plsc.subcore_barrier()

    base_l = c * _CPC + s * _PW2
    for ci, sc_hbm in enumerate((s0, s1, s2, s3, s4)):

        @pl.loop(0, _PW2 // _CHUNK)
        def _(i, sc_hbm=sc_hbm, ci=ci):
            off_l = base_l + i * _CHUNK
            off_g = ci * _CE + off_l
            pltpu.sync_copy(recv_hbm.at[pl.ds(off_g, _CHUNK)], idx_v)
            pltpu.sync_copy(sc_hbm.at[pl.ds(off_l, _CHUNK)], rows_v)
            pltpu.sync_copy(rows_v, acc_sh.at[idx_v], add=True)

    plsc.subcore_barrier()

    pltpu.sync_copy(acc_sh.at[pl.ds(z0, _ZROWS)], out_hbm.at[c, pl.ds(z0, _ZROWS)])

    @pl.when(s == _NS - 1)
    def _():
        pltpu.sync_copy(
            acc_sh.at[pl.ds(_ZROWS * _NS, _ZTAIL)],
            out_hbm.at[c, pl.ds(_ZROWS * _NS, _ZTAIL)],
        )


def _sc_scatter(scaled_chunks, receivers, zeros):
    mesh = plsc.VectorSubcoreMesh(core_axis_name="c", subcore_axis_name="s")
    f = pl.kernel(
        _sc_scatter_body,
        out_type=jax.ShapeDtypeStruct((_NC, _N, _XD), jnp.float32),
        mesh=mesh,
        scratch_types=[
            pltpu.VMEM((_CHUNK,), jnp.int32),
            pltpu.VMEM((_CHUNK, _XD), jnp.float32),
            pltpu.VMEM_SHARED((_N, _XD), jnp.float32),
        ],
        compiler_params=pltpu.CompilerParams(use_tc_tiling_on_sc=False),
    )
    return f(*scaled_chunks, receivers, zeros)


_BE = 2000


def _tc_edge_body(xs_ref, xr_ref, rbf_ref, sh_ref, cut_ref,
                  qbd_ref, kbd_ref, vbd_ref, w1_ref, b1_ref, w2_ref, b2_ref,
                  hc_ref, ec_ref, sc_ref, out_ref):
    f32 = jnp.float32
    bf16 = jnp.bfloat16
    xs = xs_ref[...]
    xr = xr_ref[...]
    xs_inv = xs[:, :_F].astype(bf16)
    xr_inv = xr[:, :_F].astype(bf16)

    dev = xs[:, _F:] - xr[:, _F:]
    inv_c = jnp.dot((dev * dev).astype(bf16), sc_ref[...], preferred_element_type=f32)

    fin = jnp.concatenate([rbf_ref[...].astype(bf16), inv_c.astype(bf16)], axis=1)
    h = jnp.dot(fin, w1_ref[...], preferred_element_type=f32) + b1_ref[...]
    h = jax.nn.silu(h)
    fwcat = jnp.dot(h.astype(bf16), w2_ref[...], preferred_element_type=f32) + b2_ref[...]

    qcat = jax.nn.silu(jnp.dot(xr_inv, qbd_ref[...], preferred_element_type=f32))
    kcat = jax.nn.silu(jnp.dot(xs_inv, kbd_ref[...], preferred_element_type=f32))
    v = jnp.dot(xs_inv, vbd_ref[...], preferred_element_type=f32)

    prod = qcat * kcat * fwcat
    alpha = jnp.dot(prod.astype(bf16), hc_ref[...], preferred_element_type=f32)
    coef = cut_ref[...] * alpha
    coef_exp = jnp.dot(coef.astype(bf16), ec_ref[...], preferred_element_type=f32)

    out_ref[:, :_F] = coef_exp[:, :_F] * v
    out_ref[:, _F:] = coef_exp[:, _F:] * sh_ref[...]


def _tc_edge(xg_s, xg_r, rbf, sh, cut, qbd, kbd, vbd, w1, b1, w2, b2, hc, ec, sc):
    grid = (_CE // _BE,)
    blk = lambda *shape: pl.BlockSpec(shape, lambda i: (i,) + (0,) * (len(shape) - 1))
    full = lambda *shape: pl.BlockSpec(shape, lambda i: (0,) * len(shape))
    return pl.pallas_call(
        _tc_edge_body,
        grid=grid,
        in_specs=[
            blk(_BE, _XD), blk(_BE, _XD), blk(_BE, 32), blk(_BE, _EV), blk(_BE, 1),
            full(_F, 256), full(_F, 256), full(_F, _F), full(36, _F), full(1, _F),
            full(_F, 256), full(1, 256), full(256, 12), full(12, _XD), full(_EV, 4),
        ],
        out_specs=blk(_BE, _XD),
        out_shape=jax.ShapeDtypeStruct((_CE, _XD), jnp.float32),
    )(xg_s, xg_r, rbf, sh, cut, qbd, kbd, vbd, w1, b1, w2, b2, hc, ec, sc)


_BN = 2000


def _tc_reduce_body(p_ref, inv_ref, ev_ref):
    s = p_ref[0] + p_ref[1]
    inv_ref[...] = s[:, :_F]
    ev_ref[...] = s[:, _F:]


def _tc_reduce(partials):
    return pl.pallas_call(
        _tc_reduce_body,
        grid=(_N // _BN,),
        in_specs=[pl.BlockSpec((2, _BN, _XD), lambda i: (0, i, 0))],
        out_specs=[
            pl.BlockSpec((_BN, _F), lambda i: (i, 0)),
            pl.BlockSpec((_BN, _EV), lambda i: (i, 0)),
        ],
        out_shape=[
            jax.ShapeDtypeStruct((_N, _F), jnp.float32),
            jax.ShapeDtypeStruct((_N, _EV), jnp.float32),
        ],
    )(partials)


def _block_diag(w):
    h, d, e = w.shape
    eye = jnp.eye(h, dtype=w.dtype)
    return jnp.einsum("hde,hg->hdge", w, eye).reshape(h * d, h * e)


def _static_mats():
    hc = np.zeros((256, 12), np.float32)
    for f in range(128):
        hc[f, f // 16] = 1.0
    for f in range(128):
        hc[128 + f, 8 + f // 32] = 1.0
    ec = np.zeros((12, _XD), np.float32)
    for h in range(8):
        ec[h, 16 * h:16 * h + 16] = 1.0 / 4.0
    for l in range(4):
        ec[8 + l, _F + _OFFS[l]:_F + _OFFS[l] + _REPEATS[l]] = 1.0 / math.sqrt(32.0)
    sc = np.zeros((_EV, 4), np.float32)
    for l in range(4):
        sc[_OFFS[l]:_OFFS[l] + _REPEATS[l], l] = 1.0
    return jnp.asarray(hc), jnp.asarray(ec), jnp.asarray(sc)


def kernel(inv_features, ev_features, rbf, senders, receivers, sh_vectors, cutoffs,
           W_q_inv, W_k_inv, W_v_inv, W_q_ev, W_k_ev,
           W1_inv, b1_inv, W2_inv, b2_inv, W1_ev, b1_ev, W2_ev, b2_ev):
    x = jnp.concatenate([inv_features, ev_features], axis=1)
    senders = senders.astype(jnp.int32)
    receivers = receivers.astype(jnp.int32)

    qbd = jnp.concatenate([_block_diag(W_q_inv), _block_diag(W_q_ev)], axis=1)
    kbd = jnp.concatenate([_block_diag(W_k_inv), _block_diag(W_k_ev)], axis=1)
    vbd = _block_diag(W_v_inv)
    w1 = jnp.concatenate([W1_inv, W1_ev], axis=1)
    b1 = jnp.concatenate([b1_inv, b1_ev]).reshape(1, -1)
    zero64 = jnp.zeros((64, _F), jnp.float32)
    w2 = jnp.concatenate(
        [jnp.concatenate([W2_inv, zero64], axis=1),
         jnp.concatenate([zero64, W2_ev], axis=1)], axis=0)
    b2 = jnp.concatenate([b2_inv, b2_ev]).reshape(1, -1)
    hc, ec, sc = _static_mats()
    bf16 = jnp.bfloat16
    qbd, kbd, vbd, w1, w2, hc, ec, sc = (
        a.astype(bf16) for a in (qbd, kbd, vbd, w1, w2, hc, ec, sc))

    scaled_chunks = []
    for ci in range(_NCK):
        lo, hi = ci * _CE, (ci + 1) * _CE
        xg_s, xg_r = _sc_gather(x, senders[lo:hi], receivers[lo:hi])
        scaled_chunks.append(_tc_edge(
            xg_s, xg_r, rbf[lo:hi], sh_vectors[lo:hi], cutoffs[lo:hi],
            qbd, kbd, vbd, w1, b1, w2, b2, hc, ec, sc))
    zeros = jnp.zeros((_ZROWS, _XD), jnp.float32)
    partials = _sc_scatter(scaled_chunks, receivers, zeros)
    d_att_inv, d_att_ev = _tc_reduce(partials)
    return (d_att_inv, d_att_ev)
